# baseline (speedup 1.0000x reference)
import functools

import jax
import jax.numpy as jnp
from jax.experimental import pallas as pl
from jax.experimental.pallas import tpu as pltpu

_LB = 10000


def _nn_kernel(u_ref, l_ref, out_ref, acc_ref, *, nblocks):
    i = pl.program_id(0)

    @pl.when(i == 0)
    def _init():
        acc_ref[:] = jnp.full_like(acc_ref, jnp.inf)

    l = l_ref[:]
    dot = jax.lax.dot_general(
        u_ref[:].astype(jnp.bfloat16), l.astype(jnp.bfloat16),
        (((1,), (1,)), ((), ())),
        preferred_element_type=jnp.float32)
    ones = jnp.ones((1, l.shape[1]), dtype=jnp.float32)
    l2 = jax.lax.dot_general(
        ones, l * l, (((1,), (1,)), ((), ())),
        preferred_element_type=jnp.float32)
    m = jnp.min(dot + l2, axis=1, keepdims=True)
    acc_ref[:] = jnp.minimum(acc_ref[:], m)

    @pl.when(i == nblocks - 1)
    def _finish():
        u = u_ref[:].astype(jnp.float32)
        u2 = 0.25 * jnp.sum(u * u, axis=1, keepdims=True)
        d = jnp.sqrt(jnp.maximum(acc_ref[:] + u2, 0.0))
        d = d - jnp.min(d)
        out_ref[:] = d / (jnp.max(d) + 1e-18)


def kernel(U_z, L_z):
    U = U_z.reshape(U_z.shape[0], -1) * -2.0
    L = L_z.reshape(L_z.shape[0], -1)
    n_u, k = U.shape
    n_l = L.shape[0]
    nblocks = n_l // _LB
    out = pl.pallas_call(
        functools.partial(_nn_kernel, nblocks=nblocks),
        grid=(nblocks,),
        in_specs=[
            pl.BlockSpec((n_u, k), lambda i: (0, 0)),
            pl.BlockSpec((_LB, k), lambda i: (i, 0)),
        ],
        out_specs=pl.BlockSpec((n_u, 1), lambda i: (0, 0)),
        out_shape=jax.ShapeDtypeStruct((n_u, 1), jnp.float32),
        scratch_shapes=[pltpu.VMEM((n_u, 1), jnp.float32)],
    )(U, L)
    return out.reshape(n_u)


# speedup vs baseline: 1.0293x; 1.0293x over previous
import functools

import jax
import jax.numpy as jnp
from jax.experimental import pallas as pl
from jax.experimental.pallas import tpu as pltpu

_LB = 10000


def _nn_kernel(u_ref, l_ref, out_ref, acc_ref, u_s, *, nblocks):
    i = pl.program_id(0)

    @pl.when(i == 0)
    def _init():
        acc_ref[:] = jnp.full_like(acc_ref, jnp.inf)
        u_s[:] = u_ref[:] * -2.0

    l = l_ref[:]
    dot = jax.lax.dot_general(
        u_s[:], l, (((1,), (1,)), ((), ())),
        preferred_element_type=jnp.float32)
    ones = jnp.ones((1, l.shape[1]), dtype=jnp.float32)
    l2 = jax.lax.dot_general(
        ones, l * l, (((1,), (1,)), ((), ())),
        preferred_element_type=jnp.float32)
    m = jnp.min(dot + l2, axis=1, keepdims=True)
    acc_ref[:] = jnp.minimum(acc_ref[:], m)

    @pl.when(i == nblocks - 1)
    def _finish():
        u = u_ref[:]
        u2 = jnp.sum(u * u, axis=1, keepdims=True)
        d = jnp.sqrt(jnp.maximum(acc_ref[:] + u2, 0.0))
        d = d - jnp.min(d)
        out_ref[:] = d / (jnp.max(d) + 1e-18)


def kernel(U_z, L_z):
    U = U_z.reshape(U_z.shape[0], -1)
    L = L_z.reshape(L_z.shape[0], -1)
    n_u, k = U.shape
    n_l = L.shape[0]
    nblocks = n_l // _LB
    out = pl.pallas_call(
        functools.partial(_nn_kernel, nblocks=nblocks),
        grid=(nblocks,),
        in_specs=[
            pl.BlockSpec((n_u, k), lambda i: (0, 0)),
            pl.BlockSpec((_LB, k), lambda i: (i, 0)),
        ],
        out_specs=pl.BlockSpec((n_u, 1), lambda i: (0, 0)),
        out_shape=jax.ShapeDtypeStruct((n_u, 1), jnp.float32),
        scratch_shapes=[pltpu.VMEM((n_u, 1), jnp.float32),
                        pltpu.VMEM((n_u, k), jnp.float32)],
    )(U, L)
    return out.reshape(n_u)


# speedup vs baseline: 1.1060x; 1.0746x over previous
import functools

import jax
import jax.numpy as jnp
from jax.experimental import pallas as pl
from jax.experimental.pallas import tpu as pltpu

_LB = 10000


def _nn_kernel(u_ref, l_ref, out_ref, acc_ref, u_s, *, nblocks):
    i = pl.program_id(0)

    @pl.when(i == 0)
    def _init():
        acc_ref[:] = jnp.full_like(acc_ref, jnp.inf)
        u_s[:, 0:128] = u_ref[:] * -2.0
        u_s[:, 128:129] = jnp.ones((u_s.shape[0], 1), jnp.float32)

    l = l_ref[:]
    l2 = jnp.sum(l * l, axis=1, keepdims=True)
    b = jnp.concatenate([l, l2], axis=1)
    t = jax.lax.dot_general(
        u_s[:], b, (((1,), (1,)), ((), ())),
        preferred_element_type=jnp.float32)
    m = jnp.min(t, axis=1, keepdims=True)
    acc_ref[:] = jnp.minimum(acc_ref[:], m)

    @pl.when(i == nblocks - 1)
    def _finish():
        u = u_ref[:]
        u2 = jnp.sum(u * u, axis=1, keepdims=True)
        d = jnp.sqrt(jnp.maximum(acc_ref[:] + u2, 0.0))
        d = d - jnp.min(d)
        out_ref[:] = d / (jnp.max(d) + 1e-18)


def kernel(U_z, L_z):
    U = U_z.reshape(U_z.shape[0], -1)
    L = L_z.reshape(L_z.shape[0], -1)
    n_u, k = U.shape
    n_l = L.shape[0]
    nblocks = n_l // _LB
    out = pl.pallas_call(
        functools.partial(_nn_kernel, nblocks=nblocks),
        grid=(nblocks,),
        in_specs=[
            pl.BlockSpec((n_u, k), lambda i: (0, 0)),
            pl.BlockSpec((_LB, k), lambda i: (i, 0)),
        ],
        out_specs=pl.BlockSpec((n_u, 1), lambda i: (0, 0)),
        out_shape=jax.ShapeDtypeStruct((n_u, 1), jnp.float32),
        scratch_shapes=[pltpu.VMEM((n_u, 1), jnp.float32),
                        pltpu.VMEM((n_u, k + 1), jnp.float32)],
    )(U, L)
    return out.reshape(n_u)


# speedup vs baseline: 1.1067x; 1.0006x over previous
import functools

import jax
import jax.numpy as jnp
from jax.experimental import pallas as pl
from jax.experimental.pallas import tpu as pltpu

_LB = 10000
_L2C = 128.0


def _nn_kernel(u_ref, l_ref, out_ref, acc_ref, u_s, *, nblocks):
    i = pl.program_id(0)

    @pl.when(i == 0)
    def _init():
        acc_ref[:] = jnp.full_like(acc_ref, jnp.inf)
        u_s[:, 0:128] = u_ref[:] * -2.0
        u_s[:, 128:129] = jnp.ones((u_s.shape[0], 1), jnp.float32)

    l = l_ref[:]
    l2 = jnp.sum(l * l, axis=1, keepdims=True) - _L2C
    b = jnp.concatenate([l, l2], axis=1)
    t = jax.lax.dot_general(
        u_s[:], b, (((1,), (1,)), ((), ())),
        preferred_element_type=jnp.float32)
    m = jnp.min(t, axis=1, keepdims=True)
    acc_ref[:] = jnp.minimum(acc_ref[:], m)

    @pl.when(i == nblocks - 1)
    def _finish():
        u = u_ref[:]
        u2 = jnp.sum(u * u, axis=1, keepdims=True) + _L2C
        d = jnp.sqrt(jnp.maximum(acc_ref[:] + u2, 0.0))
        d = d - jnp.min(d)
        out_ref[:] = d / (jnp.max(d) + 1e-18)


def kernel(U_z, L_z):
    U = U_z.reshape(U_z.shape[0], -1)
    L = L_z.reshape(L_z.shape[0], -1)
    n_u, k = U.shape
    n_l = L.shape[0]
    nblocks = n_l // _LB
    out = pl.pallas_call(
        functools.partial(_nn_kernel, nblocks=nblocks),
        grid=(nblocks,),
        in_specs=[
            pl.BlockSpec((n_u, k), lambda i: (0, 0)),
            pl.BlockSpec((_LB, k), lambda i: (i, 0)),
        ],
        out_specs=pl.BlockSpec((n_u, 1), lambda i: (0, 0)),
        out_shape=jax.ShapeDtypeStruct((n_u, 1), jnp.float32),
        scratch_shapes=[pltpu.VMEM((n_u, 1), jnp.float32),
                        pltpu.VMEM((n_u, k + 1), jnp.float32)],
    )(U, L)
    return out.reshape(n_u)


# speedup vs baseline: 1.1291x; 1.0203x over previous
import functools

import jax
import jax.numpy as jnp
from jax.experimental import pallas as pl
from jax.experimental.pallas import tpu as pltpu

_LB = 20000
_L2C = 128.0


def _nn_kernel(u_ref, l_ref, out_ref, acc_ref, u_s, *, nblocks):
    i = pl.program_id(0)

    @pl.when(i == 0)
    def _init():
        acc_ref[:] = jnp.full_like(acc_ref, jnp.inf)
        u_s[:, 0:128] = u_ref[:] * -2.0
        u_s[:, 128:129] = jnp.ones((u_s.shape[0], 1), jnp.float32)

    l = l_ref[:]
    l2 = jnp.sum(l * l, axis=1, keepdims=True) - _L2C
    b = jnp.concatenate([l, l2], axis=1)
    t = jax.lax.dot_general(
        u_s[:], b, (((1,), (1,)), ((), ())),
        preferred_element_type=jnp.float32)
    m = jnp.min(t, axis=1, keepdims=True)
    acc_ref[:] = jnp.minimum(acc_ref[:], m)

    @pl.when(i == nblocks - 1)
    def _finish():
        u = u_ref[:]
        u2 = jnp.sum(u * u, axis=1, keepdims=True) + _L2C
        d = jnp.sqrt(jnp.maximum(acc_ref[:] + u2, 0.0))
        d = d - jnp.min(d)
        out_ref[:] = d / (jnp.max(d) + 1e-18)


def kernel(U_z, L_z):
    U = U_z.reshape(U_z.shape[0], -1)
    L = L_z.reshape(L_z.shape[0], -1)
    n_u, k = U.shape
    n_l = L.shape[0]
    nblocks = n_l // _LB
    out = pl.pallas_call(
        functools.partial(_nn_kernel, nblocks=nblocks),
        grid=(nblocks,),
        in_specs=[
            pl.BlockSpec((n_u, k), lambda i: (0, 0)),
            pl.BlockSpec((_LB, k), lambda i: (i, 0)),
        ],
        out_specs=pl.BlockSpec((n_u, 1), lambda i: (0, 0)),
        out_shape=jax.ShapeDtypeStruct((n_u, 1), jnp.float32),
        scratch_shapes=[pltpu.VMEM((n_u, 1), jnp.float32),
                        pltpu.VMEM((n_u, k + 1), jnp.float32)],
    )(U, L)
    return out.reshape(n_u)


# speedup vs baseline: 1.1292x; 1.0001x over previous
import functools

import jax
import jax.numpy as jnp
from jax.experimental import pallas as pl
from jax.experimental.pallas import tpu as pltpu

_LB = 25000
_L2C = 128.0


def _nn_kernel(u_ref, l_ref, out_ref, acc_ref, u_s, *, nblocks):
    i = pl.program_id(0)

    @pl.when(i == 0)
    def _init():
        acc_ref[:] = jnp.full_like(acc_ref, jnp.inf)
        u_s[:, 0:128] = u_ref[:] * -2.0
        u_s[:, 128:129] = jnp.ones((u_s.shape[0], 1), jnp.float32)

    l = l_ref[:]
    l2 = jnp.sum(l * l, axis=1, keepdims=True) - _L2C
    b = jnp.concatenate([l, l2], axis=1)
    t = jax.lax.dot_general(
        u_s[:], b, (((1,), (1,)), ((), ())),
        preferred_element_type=jnp.float32)
    m = jnp.min(t, axis=1, keepdims=True)
    acc_ref[:] = jnp.minimum(acc_ref[:], m)

    @pl.when(i == nblocks - 1)
    def _finish():
        u = u_ref[:]
        u2 = jnp.sum(u * u, axis=1, keepdims=True) + _L2C
        d = jnp.sqrt(jnp.maximum(acc_ref[:] + u2, 0.0))
        d = d - jnp.min(d)
        out_ref[:] = d / (jnp.max(d) + 1e-18)


def kernel(U_z, L_z):
    U = U_z.reshape(U_z.shape[0], -1)
    L = L_z.reshape(L_z.shape[0], -1)
    n_u, k = U.shape
    n_l = L.shape[0]
    nblocks = n_l // _LB
    out = pl.pallas_call(
        functools.partial(_nn_kernel, nblocks=nblocks),
        grid=(nblocks,),
        in_specs=[
            pl.BlockSpec((n_u, k), lambda i: (0, 0)),
            pl.BlockSpec((_LB, k), lambda i: (i, 0)),
        ],
        out_specs=pl.BlockSpec((n_u, 1), lambda i: (0, 0)),
        out_shape=jax.ShapeDtypeStruct((n_u, 1), jnp.float32),
        scratch_shapes=[pltpu.VMEM((n_u, 1), jnp.float32),
                        pltpu.VMEM((n_u, k + 1), jnp.float32)],
    )(U, L)
    return out.reshape(n_u)
